# baseline (speedup 1.0000x reference)
import functools

import jax
import jax.numpy as jnp
from jax import lax
from jax.experimental import pallas as pl
from jax.experimental.pallas import tpu as pltpu
from jax.experimental.pallas import tpu_sc as plsc

B = 4096
S = 200
D = 32
NUM_ROWS = 100000

NC = 2
NS = 16
HALF = NUM_ROWS // NC
RPT = B // NS
CB = 2
CHUNKS = RPT // CB
NEC = CB * S
NGRP = NEC // 16
IDXC = 528
NROW = 512


def _body(x_hbm, m_hbm, w_hbm, out_hbm,
          idx_r, idx_c, msk_v, rows_v, outst_v, shared_w,
          sem_in, sem_g, sem_out):
    c = lax.axis_index("c")
    sid = lax.axis_index("s")
    base = sid * RPT
    hbase = c * HALF

    @pl.when(sid == 0)
    def _stage():
        pltpu.sync_copy(w_hbm.at[pl.ds(hbase, HALF)], shared_w)

    plsc.subcore_barrier()

    inv_s = jnp.float32(1.0 / S)
    zero16i = jnp.zeros((16,), jnp.int32)
    zero16f = jnp.zeros((16,), jnp.float32)
    lo8 = lax.iota(jnp.int32, 16) < 8
    uhalf = jnp.uint32(HALF)

    def preprocess(p):
        off = jnp.int32(0)
        s1 = off
        for grp in range(NGRP):
            e = grp * 16
            m16 = msk_v[pl.ds(e, 16)]
            iraw = idx_r[pl.ds(e, 16)]
            v = (iraw | (m16 - 1)) - hbase
            keep = v.astype(jnp.uint32) < uhalf
            if grp == 12:
                ka = keep & lo8
                plsc.store_compressed(idx_c.at[p].at[pl.ds(off, 16)], v,
                                      mask=ka)
                off = off + plsc.all_reduce_population_count(ka)[0]
                s1 = off
                kb = keep & (~lo8)
                plsc.store_compressed(idx_c.at[p].at[pl.ds(off, 16)], v,
                                      mask=kb)
                off = off + plsc.all_reduce_population_count(kb)[0]
            else:
                plsc.store_compressed(idx_c.at[p].at[pl.ds(off, 16)], v,
                                      mask=keep)
                off = off + plsc.all_reduce_population_count(keep)[0]
        for t in range(8):
            idx_c[p, pl.ds(off + t * 16, 16)] = zero16i
        return s1, off

    e00 = base * S
    pltpu.sync_copy(x_hbm.at[pl.ds(e00, NEC)], idx_r.at[pl.ds(0, NEC)])
    pltpu.sync_copy(m_hbm.at[pl.ds(e00, NEC)], msk_v.at[pl.ds(0, NEC)])
    s1_0, off_0 = preprocess(0)
    pltpu.async_copy(x_hbm.at[pl.ds(e00 + NEC, NEC)],
                     idx_r.at[pl.ds(0, NEC)], sem_in)
    pltpu.async_copy(m_hbm.at[pl.ds(e00 + NEC, NEC)],
                     msk_v.at[pl.ds(0, NEC)], sem_in)

    def chunk_body(g, carry):
        s1, off = carry
        p = g % 2
        row0 = base + g * CB
        e0 = row0 * S
        ngrp = (off + 127) // 128

        def gissue(k, carry2):
            pltpu.async_copy(shared_w.at[idx_c.at[p].at[pl.ds(k * 128, 128)]],
                             rows_v.at[pl.ds(k * 128, 128)], sem_g)
            return carry2

        lax.fori_loop(0, ngrp, gissue, 0)

        @pl.when(g < CHUNKS - 1)
        def _wait_in():
            pltpu.make_async_copy(x_hbm.at[pl.ds(e0 + NEC, NEC)],
                                  idx_r.at[pl.ds(0, NEC)], sem_in).wait()
            pltpu.make_async_copy(m_hbm.at[pl.ds(e0 + NEC, NEC)],
                                  msk_v.at[pl.ds(0, NEC)], sem_in).wait()

        s1n, offn = preprocess(1 - p)

        @pl.when(g < CHUNKS - 2)
        def _prefetch():
            en = e0 + 2 * NEC
            pltpu.async_copy(x_hbm.at[pl.ds(en, NEC)],
                             idx_r.at[pl.ds(0, NEC)], sem_in)
            pltpu.async_copy(m_hbm.at[pl.ds(en, NEC)],
                             msk_v.at[pl.ds(0, NEC)], sem_in)

        def gwait(k, carry2):
            pltpu.make_async_copy(
                shared_w.at[idx_c.at[p].at[pl.ds(k * 128, 128)]],
                rows_v.at[pl.ds(k * 128, 128)], sem_g).wait()
            return carry2

        lax.fori_loop(0, ngrp, gwait, 0)

        @pl.when(g > 0)
        def _drain():
            pltpu.make_async_copy(outst_v, out_hbm.at[c, pl.ds(row0, CB)],
                                  sem_out).wait()

        bounds = (jnp.int32(0), s1, off)
        for j in range(CB):
            lo = bounds[j]
            hi = bounds[j + 1]
            n4 = lo + ((hi - lo) // 4) * 4

            def srow4(i, accs):
                a0, a1 = accs
                t = lo + i * 4
                a0 = (a0 + rows_v[t, pl.ds(0, 16)]
                      + rows_v[t + 1, pl.ds(0, 16)]
                      + rows_v[t + 2, pl.ds(0, 16)]
                      + rows_v[t + 3, pl.ds(0, 16)])
                a1 = (a1 + rows_v[t, pl.ds(16, 16)]
                      + rows_v[t + 1, pl.ds(16, 16)]
                      + rows_v[t + 2, pl.ds(16, 16)]
                      + rows_v[t + 3, pl.ds(16, 16)])
                return a0, a1

            def srow1(t, accs):
                a0, a1 = accs
                return (a0 + rows_v[t, pl.ds(0, 16)],
                        a1 + rows_v[t, pl.ds(16, 16)])

            accs = lax.fori_loop(0, (hi - lo) // 4, srow4,
                                 (zero16f, zero16f))
            a0, a1 = lax.fori_loop(n4, hi, srow1, accs)
            outst_v[j, pl.ds(0, 16)] = a0 * inv_s
            outst_v[j, pl.ds(16, 16)] = a1 * inv_s

        pltpu.async_copy(outst_v, out_hbm.at[c, pl.ds(row0, CB)], sem_out)
        return s1n, offn

    lax.fori_loop(0, CHUNKS, chunk_body, (s1_0, off_0))
    pltpu.make_async_copy(outst_v, out_hbm.at[c, pl.ds(base, CB)],
                          sem_out).wait()


def _combine_body(p_ref, o_ref):
    o_ref[...] = p_ref[0] + p_ref[1]


@jax.jit
def _run(x, mask_i, w):
    mesh = plsc.VectorSubcoreMesh(core_axis_name="c", subcore_axis_name="s")
    f = pl.kernel(
        _body,
        out_type=jax.ShapeDtypeStruct((NC, B, D), jnp.float32),
        mesh=mesh,
        compiler_params=pltpu.CompilerParams(
            needs_layout_passes=False, use_tc_tiling_on_sc=False),
        scratch_types=[
            pltpu.VMEM((NEC,), jnp.int32),
            pltpu.VMEM((2, IDXC), jnp.int32),
            pltpu.VMEM((NEC,), jnp.int32),
            pltpu.VMEM((NROW, D), jnp.float32),
            pltpu.VMEM((CB, D), jnp.float32),
            pltpu.VMEM_SHARED((HALF, D), jnp.float32),
            pltpu.SemaphoreType.DMA,
            pltpu.SemaphoreType.DMA,
            pltpu.SemaphoreType.DMA,
        ],
    )
    partial = f(x, mask_i, w)
    return pl.pallas_call(
        _combine_body,
        out_shape=jax.ShapeDtypeStruct((B, D), jnp.float32),
    )(partial)


def kernel(x, mask, W):
    return _run(x.reshape(B * S), mask.astype(jnp.int32).reshape(B * S), W)


# speedup vs baseline: 1.1026x; 1.1026x over previous
import functools

import jax
import jax.numpy as jnp
from jax import lax
from jax.experimental import pallas as pl
from jax.experimental.pallas import tpu as pltpu
from jax.experimental.pallas import tpu_sc as plsc

B = 4096
S = 200
D = 32
NUM_ROWS = 100000

NC = 2
NS = 16
HALF = NUM_ROWS // NC
RPT = B // NS
CB = 2
CHUNKS = RPT // CB
NEC = CB * S
NGRP = NEC // 16
IDXC = 528
NROW = 512


def _body(x_hbm, w_hbm, out_hbm,
          idx_r, idx_c, rows_v, outst_v, shared_w,
          sem_in, sem_g, sem_out):
    c = lax.axis_index("c")
    sid = lax.axis_index("s")
    base = sid * RPT
    hbase = c * HALF

    @pl.when(sid == 0)
    def _stage():
        pltpu.sync_copy(w_hbm.at[pl.ds(hbase, HALF)], shared_w)

    plsc.subcore_barrier()

    inv_s = jnp.float32(1.0 / S)
    zero16i = jnp.zeros((16,), jnp.int32)
    zero16f = jnp.zeros((16,), jnp.float32)
    lo8 = lax.iota(jnp.int32, 16) < 8
    uhalf = jnp.uint32(HALF)

    e00 = base * S
    pltpu.async_copy(x_hbm.at[pl.ds(e00, NEC)], idx_r.at[pl.ds(0, NEC)],
                     sem_in)

    def chunk_body(g, carry):
        row0 = base + g * CB
        e0 = row0 * S
        pltpu.make_async_copy(x_hbm.at[pl.ds(e0, NEC)],
                              idx_r.at[pl.ds(0, NEC)], sem_in).wait()

        off = jnp.int32(0)
        bounds = [off]
        for grp in range(NGRP):
            e = grp * 16
            i16 = idx_r[pl.ds(e, 16)] - hbase
            keep = i16.astype(jnp.uint32) < uhalf
            if grp == 12:
                ka = keep & lo8
                plsc.store_compressed(idx_c.at[pl.ds(off, 16)], i16, mask=ka)
                off = off + plsc.all_reduce_population_count(ka)[0]
                bounds.append(off)
                kb = keep & (~lo8)
                plsc.store_compressed(idx_c.at[pl.ds(off, 16)], i16, mask=kb)
                off = off + plsc.all_reduce_population_count(kb)[0]
            else:
                plsc.store_compressed(idx_c.at[pl.ds(off, 16)], i16,
                                      mask=keep)
                off = off + plsc.all_reduce_population_count(keep)[0]
        bounds.append(off)

        @pl.when(g < CHUNKS - 1)
        def _prefetch():
            en = e0 + NEC
            pltpu.async_copy(x_hbm.at[pl.ds(en, NEC)],
                             idx_r.at[pl.ds(0, NEC)], sem_in)

        for t in range(8):
            idx_c[pl.ds(off + t * 16, 16)] = zero16i

        ngrp = (off + 127) // 128

        def gbody(k, carry2):
            pltpu.async_copy(shared_w.at[idx_c.at[pl.ds(k * 128, 128)]],
                             rows_v.at[pl.ds(k * 128, 128)], sem_g).wait()
            return carry2

        lax.fori_loop(0, ngrp, gbody, 0)

        @pl.when(g > 0)
        def _drain():
            pltpu.make_async_copy(outst_v, out_hbm.at[c, pl.ds(row0, CB)],
                                  sem_out).wait()

        for j in range(CB):
            lo = bounds[j]
            hi = bounds[j + 1]
            n4 = lo + ((hi - lo) // 4) * 4

            def srow4(i, accs):
                a0, a1 = accs
                t = lo + i * 4
                a0 = (a0 + rows_v[t, pl.ds(0, 16)]
                      + rows_v[t + 1, pl.ds(0, 16)]
                      + rows_v[t + 2, pl.ds(0, 16)]
                      + rows_v[t + 3, pl.ds(0, 16)])
                a1 = (a1 + rows_v[t, pl.ds(16, 16)]
                      + rows_v[t + 1, pl.ds(16, 16)]
                      + rows_v[t + 2, pl.ds(16, 16)]
                      + rows_v[t + 3, pl.ds(16, 16)])
                return a0, a1

            def srow1(t, accs):
                a0, a1 = accs
                return (a0 + rows_v[t, pl.ds(0, 16)],
                        a1 + rows_v[t, pl.ds(16, 16)])

            accs = lax.fori_loop(0, (hi - lo) // 4, srow4,
                                 (zero16f, zero16f))
            a0, a1 = lax.fori_loop(n4, hi, srow1, accs)
            outst_v[j, pl.ds(0, 16)] = a0 * inv_s
            outst_v[j, pl.ds(16, 16)] = a1 * inv_s

        pltpu.async_copy(outst_v, out_hbm.at[c, pl.ds(row0, CB)], sem_out)
        return carry

    lax.fori_loop(0, CHUNKS, chunk_body, 0)
    pltpu.make_async_copy(outst_v, out_hbm.at[c, pl.ds(base, CB)],
                          sem_out).wait()


def _combine_body(p_ref, o_ref):
    o_ref[...] = p_ref[0] + p_ref[1]


@jax.jit
def _run(x, w):
    mesh = plsc.VectorSubcoreMesh(core_axis_name="c", subcore_axis_name="s")
    f = pl.kernel(
        _body,
        out_type=jax.ShapeDtypeStruct((NC, B, D), jnp.float32),
        mesh=mesh,
        compiler_params=pltpu.CompilerParams(
            needs_layout_passes=False, use_tc_tiling_on_sc=False),
        scratch_types=[
            pltpu.VMEM((NEC,), jnp.int32),
            pltpu.VMEM((IDXC,), jnp.int32),
            pltpu.VMEM((NROW, D), jnp.float32),
            pltpu.VMEM((CB, D), jnp.float32),
            pltpu.VMEM_SHARED((HALF, D), jnp.float32),
            pltpu.SemaphoreType.DMA,
            pltpu.SemaphoreType.DMA,
            pltpu.SemaphoreType.DMA,
        ],
    )
    partial = f(x, w)
    return pl.pallas_call(
        _combine_body,
        out_shape=jax.ShapeDtypeStruct((B, D), jnp.float32),
    )(partial)


def kernel(x, mask, W):
    xm = jnp.where(mask.reshape(B * S), x.reshape(B * S), -1)
    return _run(xm, W)
